# baseline (speedup 1.0000x reference)
import functools

import jax
import jax.numpy as jnp
from jax import lax
from jax.experimental import pallas as pl
from jax.experimental.pallas import tpu as pltpu
from jax.experimental.pallas import tpu_sc as plsc

NC = 2
NS = 16
L = 16

f32 = jnp.float32
i32 = jnp.int32


def _rsqrt16(x):
  i = lax.bitcast_convert_type(x, i32)
  i = 0x5F3759DF - lax.shift_right_arithmetic(i, 1)
  y = lax.bitcast_convert_type(i, f32)
  for _ in range(3):
    y = y * (1.5 - 0.5 * x * y * y)
  return y


def _fill_vmem(ref, rows, cols, value):
  v = jnp.full((L,), value, f32)

  def body(it, _):
    r = it // (cols // L)
    k = it % (cols // L)
    ref[r, pl.ds(k * L, L)] = v
    return 0

  lax.fori_loop(0, rows * (cols // L), body, 0)


def _mm_body(x_ref, w_ref, o_ref):
  o_ref[...] = jnp.dot(x_ref[...], w_ref[...], preferred_element_type=f32)


def _matmul(x_pad, w, n_pad, d, h):
  blk = n_pad // 8
  return pl.pallas_call(
      _mm_body,
      grid=(8,),
      in_specs=[
          pl.BlockSpec((blk, d), lambda i: (i, 0)),
          pl.BlockSpec((d, h), lambda i: (0, 0)),
      ],
      out_specs=pl.BlockSpec((blk, h), lambda i: (i, 0)),
      out_shape=jax.ShapeDtypeStruct((n_pad, h), f32),
  )(x_pad, w)


def _make_mesh():
  return plsc.VectorSubcoreMesh(
      core_axis_name="c", subcore_axis_name="s", num_cores=NC,
      num_subcores=NS)


def _make_prescale(n_pad, e_pad, d):
  kb = KB
  nblk_hist = e_pad // NS // kb
  rows_w = n_pad // (NC * NS)
  mesh = _make_mesh()

  @functools.partial(
      pl.kernel,
      out_type=(
          jax.ShapeDtypeStruct((n_pad * d,), f32),
          jax.ShapeDtypeStruct((n_pad,), f32),
          jax.ShapeDtypeStruct((n_pad,), f32),
      ),
      mesh=mesh,
      scratch_types=[
          [pltpu.VMEM((kb,), i32) for _ in range(4)],
          pltpu.VMEM((kb,), f32),
          pltpu.VMEM((rows_w * d,), f32),
          pltpu.VMEM((rows_w,), f32),
          pltpu.VMEM((rows_w,), f32),
          pltpu.VMEM((rows_w,), f32),
          pltpu.VMEM_SHARED((n_pad,), f32),
          [pltpu.SemaphoreType.DMA for _ in range(4)],
          [pltpu.SemaphoreType.DMA for _ in range(4)],
      ],
  )
  def prescale(col_hbm, xw_hbm, u_hbm, d1_hbm, d2_hbm,
               cidx, ones_v, xwb, degv, d1v, d2v, deg_sh, isem, ssem):
    c = lax.axis_index("c")
    s = lax.axis_index("s")
    wid = s * NC + c

    nz = n_pad // NS
    def zfill(it, _):
      degv[pl.ds(it * L, L)] = jnp.zeros((L,), f32)
      return 0
    lax.fori_loop(0, rows_w // L, zfill, 0)
    for rep in range(nz // rows_w):
      pltpu.sync_copy(degv, deg_sh.at[pl.ds(s * nz + rep * rows_w, rows_w)])
    plsc.subcore_barrier()

    def ofill(it, _):
      ones_v[pl.ds(it * L, L)] = jnp.ones((L,), f32)
      return 0
    lax.fori_loop(0, kb // L, ofill, 0)
    ebase = s * (nblk_hist * kb)

    def stage(j, b):
      pltpu.async_copy(col_hbm.at[pl.ds(ebase + j * kb, kb)], cidx[b],
                       isem[b])

    def wait_stage(b):
      pltpu.make_async_copy(col_hbm.at[pl.ds(0, kb)], cidx[b],
                            isem[b]).wait()

    def hscat(b):
      pltpu.async_copy(ones_v, deg_sh.at[cidx[b]], ssem[b], add=True)

    def wait_hscat(b):
      pltpu.make_async_copy(ones_v, deg_sh.at[cidx[0]], ssem[b]).wait()

    for b in range(3):
      stage(b, b)

    def hbody(g, _):
      for b in range(4):
        j = g * 4 + b
        wait_stage(b)
        hscat(b)
        bq = (b + 3) % 4

        @pl.when(j - 1 >= 0)
        def _():
          wait_hscat(bq)

        @pl.when(j + 3 < nblk_hist)
        def _():
          stage(j + 3, bq)
      return 0
    lax.fori_loop(0, nblk_hist // 4, hbody, 0)
    wait_hscat((nblk_hist - 1) % 4)
    plsc.subcore_barrier()

    base_r = wid * rows_w
    pltpu.sync_copy(deg_sh.at[pl.ds(base_r, rows_w)], degv)
    pltpu.sync_copy(xw_hbm.at[pl.ds(base_r * d, rows_w * d)], xwb)

    def dbody(i, _):
      dg = degv[pl.ds(i * L, L)]
      d1 = _rsqrt16(dg + 1.0)
      d2 = jnp.where(dg > 0.0, _rsqrt16(jnp.maximum(dg, 1.0)), 0.0)
      d1v[pl.ds(i * L, L)] = d1
      d2v[pl.ds(i * L, L)] = d2
      return 0
    lax.fori_loop(0, rows_w // L, dbody, 0)
    pltpu.sync_copy(d1v, d1_hbm.at[pl.ds(base_r, rows_w)])
    pltpu.sync_copy(d2v, d2_hbm.at[pl.ds(base_r, rows_w)])

    def gbody(g, _):
      dv1 = d1v[pl.ds(g * L, L)]
      for r16 in range(L):
        sc = jnp.zeros((L,), f32) + dv1[r16]
        off = g * (L * d) + r16 * d
        for k in range(d // L):
          sl = pl.ds(off + k * L, L)
          xwb[sl] = xwb[sl] * sc
      return 0
    lax.fori_loop(0, rows_w // L, gbody, 0)
    pltpu.sync_copy(xwb, u_hbm.at[pl.ds(base_r * d, rows_w * d)])

  return prescale


KB = 56
NBR = 6
NBI = 12
WS = 3


def _make_propagate(n_pad, e_pad, d):
  kb, nbr, nbi, w = KB, NBR, NBI, WS
  nblk_t = e_pad // NS // kb
  nb0 = (nblk_t // 2) // nbi * nbi
  nb1 = nblk_t - nb0
  assert nb1 % nbi == 0 and nb0 >= nbi and nb1 >= nbi
  for k in range(1, w + 1):
    assert (nb0 - k) % nbr == (nb1 - k) % nbr
  mesh = _make_mesh()

  @functools.partial(
      pl.kernel,
      out_type=jax.ShapeDtypeStruct((NC * n_pad, d), f32),
      mesh=mesh,
      scratch_types=[
          [pltpu.VMEM((2, kb), i32) for _ in range(nbi)],
          [pltpu.VMEM((kb, d), f32) for _ in range(nbr)],
          pltpu.VMEM_SHARED((n_pad, d), f32),
          [pltpu.SemaphoreType.DMA for _ in range(nbi)],
          [pltpu.SemaphoreType.DMA for _ in range(nbr)],
          [pltpu.SemaphoreType.DMA for _ in range(nbr)],
      ],
  )
  def propagate(rc_hbm, src_hbm, out_hbm,
                idxs, rows, acc_sh, isem, gsem, ssem):
    c = lax.axis_index("c")
    s = lax.axis_index("s")

    _fill_vmem(rows[0], kb, d, 0.0)
    nz = n_pad // NS
    for rep in range(nz // kb):
      pltpu.sync_copy(rows[0], acc_sh.at[pl.ds(s * nz + rep * kb, kb)])
    rem = nz % kb
    if rem:
      pltpu.sync_copy(rows[0].at[pl.ds(0, rem)],
                      acc_sh.at[pl.ds(s * nz + (nz // kb) * kb, rem)])

    nb = jnp.where(c == 0, nb0, nb1)
    ng = jnp.where(c == 0, nb0 // nbi, nb1 // nbi)
    base_b = jnp.where(c == 0, s * nb0, NS * nb0 + s * nb1)

    def stage(j, bi):
      pltpu.async_copy(rc_hbm.at[base_b + j], idxs[bi], isem[bi])

    def wait_stage(bi):
      pltpu.make_async_copy(rc_hbm.at[0], idxs[bi], isem[bi]).wait()

    def gather(br, bi):
      pltpu.async_copy(src_hbm.at[idxs[bi].at[0]], rows[br], gsem[br])

    def wait_gather(br):
      pltpu.make_async_copy(src_hbm.at[idxs[0].at[0]], rows[br],
                            gsem[br]).wait()

    def scatter(br, bi):
      pltpu.async_copy(rows[br], acc_sh.at[idxs[bi].at[1]], ssem[br],
                       add=True)

    def wait_scatter(br):
      pltpu.make_async_copy(rows[br], acc_sh.at[idxs[0].at[1]],
                            ssem[br]).wait()

    plsc.subcore_barrier()

    for b in range(nbi - w):
      stage(b, b)
    for b in range(nbr - w):
      wait_stage(b)
      gather(b, b)

    def gbody(g, _):
      for b in range(nbi):
        j = g * nbi + b
        br = b % nbr
        wait_gather(br)
        scatter(br, b)
        brq = (br + nbr - w) % nbr
        biq = (b + nbi - w) % nbi

        @pl.when(j - w >= 0)
        def _():
          wait_scatter(brq)

        @pl.when(j + nbi - w < nb)
        def _():
          stage(j + nbi - w, biq)

        bi_g = (b + nbr - w) % nbi

        @pl.when(j + nbr - w < nb)
        def _():
          wait_stage(bi_g)
          gather(brq, bi_g)
      return 0
    lax.fori_loop(0, ng, gbody, 0)

    for k in range(w, 0, -1):
      wait_scatter((nb0 - k) % nbr)
    plsc.subcore_barrier()

    nd = n_pad // NS
    pltpu.sync_copy(acc_sh.at[pl.ds(s * nd, nd)],
                    out_hbm.at[pl.ds(c * n_pad + s * nd, nd)])

  return propagate


def _make_combine1(n_pad, d):
  rows_w = n_pad // (NC * NS)
  cb = 64
  mesh = _make_mesh()

  @functools.partial(
      pl.kernel,
      out_type=(
          jax.ShapeDtypeStruct((n_pad * d,), f32),
          jax.ShapeDtypeStruct((n_pad * d,), f32),
      ),
      mesh=mesh,
      scratch_types=[
          pltpu.VMEM((cb * d,), f32),
          pltpu.VMEM((cb * d,), f32),
          pltpu.VMEM((cb * d,), f32),
          pltpu.VMEM((rows_w,), f32),
          pltpu.VMEM((rows_w,), f32),
          pltpu.VMEM((L,), f32),
      ],
  )
  def combine1(p_hbm, xw_hbm, d1_hbm, d2_hbm, a0_hbm, v_hbm, r0_hbm,
               pa, pb, xwb, d1v, d2v, a0v):
    c = lax.axis_index("c")
    s = lax.axis_index("s")
    wid = s * NC + c
    base_r = wid * rows_w
    npd = n_pad * d
    pltpu.sync_copy(d1_hbm.at[pl.ds(base_r, rows_w)], d1v)
    pltpu.sync_copy(d2_hbm.at[pl.ds(base_r, rows_w)], d2v)
    pltpu.sync_copy(a0_hbm, a0v)

    def cbody(ch, _):
      boff = pl.multiple_of((base_r + ch * cb) * d, cb * d)
      pltpu.sync_copy(p_hbm.at[pl.ds(boff, cb * d)], pa)
      pltpu.sync_copy(p_hbm.at[pl.ds(npd + boff, cb * d)], pb)
      pltpu.sync_copy(xw_hbm.at[pl.ds(boff, cb * d)], xwb)

      def gbody(g, _):
        dbase = ch * cb + g * L
        dv1 = d1v[pl.ds(dbase, L)]
        dv2 = d2v[pl.ds(dbase, L)]
        a0 = a0v[pl.ds(0, L)]
        for r16 in range(L):
          sd1 = jnp.zeros((L,), f32) + dv1[r16]
          sd2 = jnp.zeros((L,), f32) + dv2[r16]
          off = g * (L * d) + r16 * d
          for k in range(d // L):
            sl = pl.ds(off + k * L, L)
            hrow = sd1 * (pa[sl] + pb[sl]) + xwb[sl] * sd1 * sd1
            xwb[sl] = hrow * sd2
            pb[sl] = hrow * a0
        return 0
      lax.fori_loop(0, cb // L, gbody, 0)
      pltpu.sync_copy(xwb, v_hbm.at[pl.ds(boff, cb * d)])
      pltpu.sync_copy(pb, r0_hbm.at[pl.ds(boff, cb * d)])
      return 0
    lax.fori_loop(0, rows_w // cb, cbody, 0)

  return combine1


def _make_combine2(n_pad, d):
  rows_w = n_pad // (NC * NS)
  cb = 64
  mesh = _make_mesh()

  @functools.partial(
      pl.kernel,
      out_type=jax.ShapeDtypeStruct((n_pad * d,), f32),
      mesh=mesh,
      scratch_types=[
          pltpu.VMEM((cb * d,), f32),
          pltpu.VMEM((cb * d,), f32),
          pltpu.VMEM((cb * d,), f32),
          pltpu.VMEM((rows_w,), f32),
          pltpu.VMEM((L,), f32),
      ],
  )
  def combine2(q_hbm, r0_hbm, d2_hbm, a1_hbm, res_hbm,
               qa, qb, r0b, d2v, a1v):
    c = lax.axis_index("c")
    s = lax.axis_index("s")
    wid = s * NC + c
    base_r = wid * rows_w
    npd = n_pad * d
    pltpu.sync_copy(d2_hbm.at[pl.ds(base_r, rows_w)], d2v)
    pltpu.sync_copy(a1_hbm, a1v)

    def cbody(ch, _):
      boff = pl.multiple_of((base_r + ch * cb) * d, cb * d)
      pltpu.sync_copy(q_hbm.at[pl.ds(boff, cb * d)], qa)
      pltpu.sync_copy(q_hbm.at[pl.ds(npd + boff, cb * d)], qb)
      pltpu.sync_copy(r0_hbm.at[pl.ds(boff, cb * d)], r0b)

      def gbody(g, _):
        dbase = ch * cb + g * L
        dv2 = d2v[pl.ds(dbase, L)]
        a1 = a1v[pl.ds(0, L)]
        for r16 in range(L):
          sd2 = jnp.zeros((L,), f32) + dv2[r16]
          off = g * (L * d) + r16 * d
          for k in range(d // L):
            sl = pl.ds(off + k * L, L)
            qb[sl] = r0b[sl] + a1 * sd2 * (qa[sl] + qb[sl])
        return 0
      lax.fori_loop(0, cb // L, gbody, 0)
      pltpu.sync_copy(qb, res_hbm.at[pl.ds(boff, cb * d)])
      return 0
    lax.fori_loop(0, rows_w // cb, cbody, 0)

  return combine2


def kernel(edge_index, in_feat, W, alphas):
  n, d = in_feat.shape
  h = W.shape[1]
  e = edge_index.shape[1]

  n_pad = ((n + (NC * NS * 16) - 1) // (NC * NS * 16)) * (NC * NS * 16)
  eblk = NS * KB * NBI
  e_pad = ((e + eblk - 1) // eblk) * eblk

  row = edge_index[0].astype(i32)
  col = edge_index[1].astype(i32)
  if n_pad == n:
    n_pad += NC * NS * 16
  pad_e = e_pad - e
  dummies = n + (jnp.arange(pad_e, dtype=i32) % (n_pad - n))
  row_p = jnp.concatenate([row, dummies])
  col_p = jnp.concatenate([col, dummies])
  rc = jnp.stack([row_p.reshape(e_pad // KB, KB),
                  col_p.reshape(e_pad // KB, KB)], axis=1)
  x_pad = jnp.zeros((n_pad, d), f32).at[:n].set(in_feat)

  alpha = jax.nn.softmax(alphas.astype(f32), axis=0)
  a0 = jnp.full((L,), alpha[0], f32)
  a1 = jnp.full((L,), alpha[1], f32)

  xw = _matmul(x_pad, W.astype(f32), n_pad, d, h)
  xw_flat = xw.reshape(-1)

  prop = _make_propagate(n_pad, e_pad, h)
  u, d1, d2 = _make_prescale(n_pad, e_pad, h)(col_p, xw_flat)
  p = prop(rc, u.reshape(n_pad, h))
  v, r0 = _make_combine1(n_pad, h)(p.reshape(-1), xw_flat, d1, d2, a0)
  q = prop(rc, v.reshape(n_pad, h))
  res = _make_combine2(n_pad, h)(q.reshape(-1), r0, d2, a1)
  return res.reshape(n_pad, h)[:n]


# speedup vs baseline: 1.1426x; 1.1426x over previous
import functools

import jax
import jax.numpy as jnp
from jax import lax
from jax.experimental import pallas as pl
from jax.experimental.pallas import tpu as pltpu
from jax.experimental.pallas import tpu_sc as plsc

NC = 2
NS = 16
L = 16

f32 = jnp.float32
i32 = jnp.int32


def _rsqrt16(x):
  i = lax.bitcast_convert_type(x, i32)
  i = 0x5F3759DF - lax.shift_right_arithmetic(i, 1)
  y = lax.bitcast_convert_type(i, f32)
  for _ in range(3):
    y = y * (1.5 - 0.5 * x * y * y)
  return y


def _fill_vmem(ref, rows, cols, value):
  v = jnp.full((L,), value, f32)

  def body(it, _):
    r = it // (cols // L)
    k = it % (cols // L)
    ref[r, pl.ds(k * L, L)] = v
    return 0

  lax.fori_loop(0, rows * (cols // L), body, 0)


def _mm_body(x_ref, w_ref, o_ref):
  o_ref[...] = jnp.dot(x_ref[...], w_ref[...], preferred_element_type=f32)


def _matmul(x_pad, w, n_pad, d, h):
  blk = n_pad // 8
  return pl.pallas_call(
      _mm_body,
      grid=(8,),
      in_specs=[
          pl.BlockSpec((blk, d), lambda i: (i, 0)),
          pl.BlockSpec((d, h), lambda i: (0, 0)),
      ],
      out_specs=pl.BlockSpec((blk, h), lambda i: (i, 0)),
      out_shape=jax.ShapeDtypeStruct((n_pad, h), f32),
  )(x_pad, w)


def _make_mesh():
  return plsc.VectorSubcoreMesh(
      core_axis_name="c", subcore_axis_name="s", num_cores=NC,
      num_subcores=NS)


def _make_hist(n_pad, e_pad):
  kb = 112
  nblk_hist = e_pad // NS // kb
  rows_w = n_pad // (NC * NS)
  mesh = _make_mesh()

  @functools.partial(
      pl.kernel,
      out_type=(
          jax.ShapeDtypeStruct((n_pad,), f32),
          jax.ShapeDtypeStruct((n_pad,), f32),
      ),
      mesh=mesh,
      scratch_types=[
          [pltpu.VMEM((kb,), i32) for _ in range(4)],
          pltpu.VMEM((kb,), f32),
          pltpu.VMEM((rows_w,), f32),
          pltpu.VMEM((rows_w,), f32),
          pltpu.VMEM((rows_w,), f32),
          pltpu.VMEM_SHARED((n_pad,), f32),
          [pltpu.SemaphoreType.DMA for _ in range(4)],
          [pltpu.SemaphoreType.DMA for _ in range(4)],
      ],
  )
  def hist(col_hbm, d1_hbm, d2_hbm,
           cidx, ones_v, degv, d1v, d2v, deg_sh, isem, ssem):
    c = lax.axis_index("c")
    s = lax.axis_index("s")
    wid = s * NC + c

    nz = n_pad // NS
    def zfill(it, _):
      degv[pl.ds(it * L, L)] = jnp.zeros((L,), f32)
      return 0
    lax.fori_loop(0, rows_w // L, zfill, 0)
    for rep in range(nz // rows_w):
      pltpu.sync_copy(degv, deg_sh.at[pl.ds(s * nz + rep * rows_w, rows_w)])
    plsc.subcore_barrier()

    def ofill(it, _):
      ones_v[pl.ds(it * L, L)] = jnp.ones((L,), f32)
      return 0
    lax.fori_loop(0, kb // L, ofill, 0)
    ebase = s * (nblk_hist * kb)

    def stage(j, b):
      pltpu.async_copy(col_hbm.at[pl.ds(ebase + j * kb, kb)], cidx[b],
                       isem[b])

    def wait_stage(b):
      pltpu.make_async_copy(col_hbm.at[pl.ds(0, kb)], cidx[b],
                            isem[b]).wait()

    def hscat(b):
      pltpu.async_copy(ones_v, deg_sh.at[cidx[b]], ssem[b], add=True)

    def wait_hscat(b):
      pltpu.make_async_copy(ones_v, deg_sh.at[cidx[0]], ssem[b]).wait()

    for b in range(3):
      stage(b, b)

    def hbody(g, _):
      for b in range(4):
        j = g * 4 + b
        wait_stage(b)
        hscat(b)
        bq = (b + 3) % 4

        @pl.when(j - 1 >= 0)
        def _():
          wait_hscat(bq)

        @pl.when(j + 3 < nblk_hist)
        def _():
          stage(j + 3, bq)
      return 0
    lax.fori_loop(0, nblk_hist // 4, hbody, 0)
    wait_hscat((nblk_hist - 1) % 4)
    plsc.subcore_barrier()

    base_r = wid * rows_w
    pltpu.sync_copy(deg_sh.at[pl.ds(base_r, rows_w)], degv)

    def dbody(i, _):
      dg = degv[pl.ds(i * L, L)]
      d1 = _rsqrt16(dg + 1.0)
      d2 = jnp.where(dg > 0.0, _rsqrt16(jnp.maximum(dg, 1.0)), 0.0)
      d1v[pl.ds(i * L, L)] = d1
      d2v[pl.ds(i * L, L)] = d2
      return 0
    lax.fori_loop(0, rows_w // L, dbody, 0)
    pltpu.sync_copy(d1v, d1_hbm.at[pl.ds(base_r, rows_w)])
    pltpu.sync_copy(d2v, d2_hbm.at[pl.ds(base_r, rows_w)])

  return hist


def _make_scale(n_pad, d):
  rows_w = n_pad // (NC * NS)
  mesh = _make_mesh()

  @functools.partial(
      pl.kernel,
      out_type=jax.ShapeDtypeStruct((n_pad * d,), f32),
      mesh=mesh,
      scratch_types=[
          pltpu.VMEM((rows_w * d,), f32),
          pltpu.VMEM((rows_w,), f32),
      ],
  )
  def scale(xw_hbm, d1_hbm, u_hbm, xwb, d1v):
    c = lax.axis_index("c")
    s = lax.axis_index("s")
    wid = s * NC + c
    base_r = wid * rows_w
    pltpu.sync_copy(d1_hbm.at[pl.ds(base_r, rows_w)], d1v)
    pltpu.sync_copy(xw_hbm.at[pl.ds(base_r * d, rows_w * d)], xwb)

    def gbody(g, _):
      dv1 = d1v[pl.ds(g * L, L)]
      for r16 in range(L):
        sc = jnp.zeros((L,), f32) + dv1[r16]
        off = g * (L * d) + r16 * d
        for k in range(d // L):
          sl = pl.ds(off + k * L, L)
          xwb[sl] = xwb[sl] * sc
      return 0
    lax.fori_loop(0, rows_w // L, gbody, 0)
    pltpu.sync_copy(xwb, u_hbm.at[pl.ds(base_r * d, rows_w * d)])

  return scale


KB = 112
NBR = 3
NBI = 6
WS = 1


def _make_propagate(n_pad, e_pad, d):
  kb, nbr, nbi, w = KB, NBR, NBI, WS
  nblk_t = e_pad // NS // kb
  nb0 = (nblk_t // 2) // nbi * nbi
  nb1 = nblk_t - nb0
  assert nb1 % nbi == 0 and nb0 >= nbi and nb1 >= nbi
  for k in range(1, w + 1):
    assert (nb0 - k) % nbr == (nb1 - k) % nbr
  mesh = _make_mesh()

  @functools.partial(
      pl.kernel,
      out_type=jax.ShapeDtypeStruct((NC * n_pad, d), f32),
      mesh=mesh,
      scratch_types=[
          [pltpu.VMEM((2, kb), i32) for _ in range(nbi)],
          [pltpu.VMEM((kb, d), f32) for _ in range(nbr)],
          pltpu.VMEM_SHARED((n_pad, d), f32),
          [pltpu.SemaphoreType.DMA for _ in range(nbi)],
          [pltpu.SemaphoreType.DMA for _ in range(nbr)],
          [pltpu.SemaphoreType.DMA for _ in range(nbr)],
      ],
  )
  def propagate(rc_hbm, src_hbm, out_hbm,
                idxs, rows, acc_sh, isem, gsem, ssem):
    c = lax.axis_index("c")
    s = lax.axis_index("s")

    _fill_vmem(rows[0], kb, d, 0.0)
    nz = n_pad // NS
    for rep in range(nz // kb):
      pltpu.sync_copy(rows[0], acc_sh.at[pl.ds(s * nz + rep * kb, kb)])
    rem = nz % kb
    if rem:
      pltpu.sync_copy(rows[0].at[pl.ds(0, rem)],
                      acc_sh.at[pl.ds(s * nz + (nz // kb) * kb, rem)])

    nb = jnp.where(c == 0, nb0, nb1)
    ng = jnp.where(c == 0, nb0 // nbi, nb1 // nbi)
    base_b = jnp.where(c == 0, s * nb0, NS * nb0 + s * nb1)

    def stage(j, bi):
      pltpu.async_copy(rc_hbm.at[base_b + j], idxs[bi], isem[bi])

    def wait_stage(bi):
      pltpu.make_async_copy(rc_hbm.at[0], idxs[bi], isem[bi]).wait()

    def gather(br, bi):
      pltpu.async_copy(src_hbm.at[idxs[bi].at[0]], rows[br], gsem[br])

    def wait_gather(br):
      pltpu.make_async_copy(src_hbm.at[idxs[0].at[0]], rows[br],
                            gsem[br]).wait()

    def scatter(br, bi):
      pltpu.async_copy(rows[br], acc_sh.at[idxs[bi].at[1]], ssem[br],
                       add=True)

    def wait_scatter(br):
      pltpu.make_async_copy(rows[br], acc_sh.at[idxs[0].at[1]],
                            ssem[br]).wait()

    plsc.subcore_barrier()

    for b in range(nbi - w):
      stage(b, b)
    for b in range(nbr - w):
      wait_stage(b)
      gather(b, b)

    def gbody(g, _):
      for b in range(nbi):
        j = g * nbi + b
        br = b % nbr
        wait_gather(br)
        scatter(br, b)
        brq = (br + nbr - w) % nbr
        biq = (b + nbi - w) % nbi

        @pl.when(j - w >= 0)
        def _():
          wait_scatter(brq)

        @pl.when(j + nbi - w < nb)
        def _():
          stage(j + nbi - w, biq)

        bi_g = (b + nbr - w) % nbi

        @pl.when(j + nbr - w < nb)
        def _():
          wait_stage(bi_g)
          gather(brq, bi_g)
      return 0
    lax.fori_loop(0, ng, gbody, 0)

    for k in range(w, 0, -1):
      wait_scatter((nb0 - k) % nbr)
    plsc.subcore_barrier()

    nd = n_pad // NS
    pltpu.sync_copy(acc_sh.at[pl.ds(s * nd, nd)],
                    out_hbm.at[pl.ds(c * n_pad + s * nd, nd)])

  return propagate


def _make_combine1(n_pad, d):
  rows_w = n_pad // (NC * NS)
  cb = 64
  mesh = _make_mesh()

  @functools.partial(
      pl.kernel,
      out_type=(
          jax.ShapeDtypeStruct((n_pad * d,), f32),
          jax.ShapeDtypeStruct((n_pad * d,), f32),
      ),
      mesh=mesh,
      scratch_types=[
          pltpu.VMEM((cb * d,), f32),
          pltpu.VMEM((cb * d,), f32),
          pltpu.VMEM((cb * d,), f32),
          pltpu.VMEM((rows_w,), f32),
          pltpu.VMEM((rows_w,), f32),
          pltpu.VMEM((L,), f32),
      ],
  )
  def combine1(p_hbm, xw_hbm, d1_hbm, d2_hbm, a0_hbm, v_hbm, r0_hbm,
               pa, pb, xwb, d1v, d2v, a0v):
    c = lax.axis_index("c")
    s = lax.axis_index("s")
    wid = s * NC + c
    base_r = wid * rows_w
    npd = n_pad * d
    pltpu.sync_copy(d1_hbm.at[pl.ds(base_r, rows_w)], d1v)
    pltpu.sync_copy(d2_hbm.at[pl.ds(base_r, rows_w)], d2v)
    pltpu.sync_copy(a0_hbm, a0v)

    def cbody(ch, _):
      boff = pl.multiple_of((base_r + ch * cb) * d, cb * d)
      pltpu.sync_copy(p_hbm.at[pl.ds(boff, cb * d)], pa)
      pltpu.sync_copy(p_hbm.at[pl.ds(npd + boff, cb * d)], pb)
      pltpu.sync_copy(xw_hbm.at[pl.ds(boff, cb * d)], xwb)

      def gbody(g, _):
        dbase = ch * cb + g * L
        dv1 = d1v[pl.ds(dbase, L)]
        dv2 = d2v[pl.ds(dbase, L)]
        a0 = a0v[pl.ds(0, L)]
        for r16 in range(L):
          sd1 = jnp.zeros((L,), f32) + dv1[r16]
          sd2 = jnp.zeros((L,), f32) + dv2[r16]
          off = g * (L * d) + r16 * d
          for k in range(d // L):
            sl = pl.ds(off + k * L, L)
            hrow = sd1 * (pa[sl] + pb[sl]) + xwb[sl] * sd1 * sd1
            xwb[sl] = hrow * sd2
            pb[sl] = hrow * a0
        return 0
      lax.fori_loop(0, cb // L, gbody, 0)
      pltpu.sync_copy(xwb, v_hbm.at[pl.ds(boff, cb * d)])
      pltpu.sync_copy(pb, r0_hbm.at[pl.ds(boff, cb * d)])
      return 0
    lax.fori_loop(0, rows_w // cb, cbody, 0)

  return combine1


def _make_combine2(n_pad, d, n_real):
  rows_w = n_pad // (NC * NS)
  cb = 64
  rem = n_real % cb
  mesh = _make_mesh()

  @functools.partial(
      pl.kernel,
      out_type=jax.ShapeDtypeStruct((n_real * d,), f32),
      mesh=mesh,
      scratch_types=[
          pltpu.VMEM((cb * d,), f32),
          pltpu.VMEM((cb * d,), f32),
          pltpu.VMEM((cb * d,), f32),
          pltpu.VMEM((rows_w,), f32),
          pltpu.VMEM((L,), f32),
      ],
  )
  def combine2(q_hbm, r0_hbm, d2_hbm, a1_hbm, res_hbm,
               qa, qb, r0b, d2v, a1v):
    c = lax.axis_index("c")
    s = lax.axis_index("s")
    wid = s * NC + c
    base_r = wid * rows_w
    npd = n_pad * d
    pltpu.sync_copy(d2_hbm.at[pl.ds(base_r, rows_w)], d2v)
    pltpu.sync_copy(a1_hbm, a1v)

    def cbody(ch, _):
      boff = pl.multiple_of((base_r + ch * cb) * d, cb * d)
      pltpu.sync_copy(q_hbm.at[pl.ds(boff, cb * d)], qa)
      pltpu.sync_copy(q_hbm.at[pl.ds(npd + boff, cb * d)], qb)
      pltpu.sync_copy(r0_hbm.at[pl.ds(boff, cb * d)], r0b)

      def gbody(g, _):
        dbase = ch * cb + g * L
        dv2 = d2v[pl.ds(dbase, L)]
        a1 = a1v[pl.ds(0, L)]
        for r16 in range(L):
          sd2 = jnp.zeros((L,), f32) + dv2[r16]
          off = g * (L * d) + r16 * d
          for k in range(d // L):
            sl = pl.ds(off + k * L, L)
            qb[sl] = r0b[sl] + a1 * sd2 * (qa[sl] + qb[sl])
        return 0
      lax.fori_loop(0, cb // L, gbody, 0)

      @pl.when(boff + cb * d <= n_real * d)
      def _():
        pltpu.sync_copy(qb, res_hbm.at[pl.ds(boff, cb * d)])

      if rem:
        @pl.when(jnp.logical_and(boff < n_real * d,
                                 boff + cb * d > n_real * d))
        def _():
          pltpu.sync_copy(qb.at[pl.ds(0, rem * d)],
                          res_hbm.at[pl.ds(boff, rem * d)])
      return 0
    lax.fori_loop(0, rows_w // cb, cbody, 0)

  return combine2


def kernel(edge_index, in_feat, W, alphas):
  n, d = in_feat.shape
  h = W.shape[1]
  e = edge_index.shape[1]

  n_pad = ((n + (NC * NS * 16) - 1) // (NC * NS * 16)) * (NC * NS * 16)
  eblk = NS * KB * NBI * 2
  e_pad = ((e + eblk - 1) // eblk) * eblk

  row = edge_index[0].astype(i32)
  col = edge_index[1].astype(i32)
  if n_pad == n:
    n_pad += NC * NS * 16
  pad_e = e_pad - e
  dummies = n + (jnp.arange(pad_e, dtype=i32) % (n_pad - n))
  row_p = jnp.concatenate([row, dummies])
  col_p = jnp.concatenate([col, dummies])
  rc = jnp.stack([row_p.reshape(e_pad // KB, KB),
                  col_p.reshape(e_pad // KB, KB)], axis=1)
  x_pad = jnp.zeros((n_pad, d), f32).at[:n].set(in_feat)

  alpha = jax.nn.softmax(alphas.astype(f32), axis=0)
  a0 = jnp.full((L,), alpha[0], f32)
  a1 = jnp.full((L,), alpha[1], f32)

  xw = _matmul(x_pad, W.astype(f32), n_pad, d, h)
  xw_flat = xw.reshape(-1)

  prop = _make_propagate(n_pad, e_pad, h)
  d1, d2 = _make_hist(n_pad, e_pad)(col_p)
  u = _make_scale(n_pad, h)(xw_flat, d1)
  p = prop(rc, u.reshape(n_pad, h))
  v, r0 = _make_combine1(n_pad, h)(p.reshape(-1), xw_flat, d1, d2, a0)
  q = prop(rc, v.reshape(n_pad, h))
  res = _make_combine2(n_pad, h, n)(q.reshape(-1), r0, d2, a1)
  return res.reshape(n, h)
